# baseline (speedup 1.0000x reference)
import functools

import jax
import jax.numpy as jnp
from jax import lax
from jax.experimental import pallas as pl
from jax.experimental.pallas import tpu as pltpu
from jax.experimental.pallas import tpu_sc as plsc

ALPHA = 40.0
N_EPS = 1e-6
L = 16


def _sqrt16(x):
  xs = jnp.maximum(x, jnp.float32(1e-30))
  i = plsc.bitcast(xs, jnp.int32)
  i = jnp.int32(0x5F3759DF) - (i >> 1)
  y = plsc.bitcast(i, jnp.float32)
  for _ in range(3):
    y = y * (jnp.float32(1.5) - jnp.float32(0.5) * xs * y * y)
  return xs * y


def _tc_exp_kernel(n, block_rows):
  grid = n // block_rows

  def body(ga_ref, gb_ref, pa_ref, pb_ref, eg_ref, ep_ref):
    dg = ga_ref[...] - gb_ref[...]
    dp = pa_ref[...] - pb_ref[...]
    sg = jnp.sum(dg * dg, axis=1)
    sp = jnp.sum(dp * dp, axis=1)
    eg_ref[...] = jnp.exp(-jnp.float32(ALPHA) * jnp.sqrt(sg))
    ep_ref[...] = jnp.exp(-jnp.float32(ALPHA) * jnp.sqrt(sp))

  in_spec = pl.BlockSpec((block_rows, 3), lambda i: (i, 0))
  out_spec = pl.BlockSpec((block_rows,), lambda i: (i,))
  return pl.pallas_call(
      body,
      grid=(grid,),
      in_specs=[in_spec] * 4,
      out_specs=[out_spec, out_spec],
      out_shape=[
          jax.ShapeDtypeStruct((n,), jnp.float32),
          jax.ShapeDtypeStruct((n,), jnp.float32),
      ],
  )


def _sc_loss_kernel(n, num_subcores):
  n_per_tile = n // num_subcores
  groups = n_per_tile // L
  mesh = plsc.VectorSubcoreMesh(
      core_axis_name="c", subcore_axis_name="s",
      num_cores=2, num_subcores=num_subcores)
  inv_scale = jnp.float32(0.5 / n)

  @functools.partial(
      pl.kernel,
      out_type=jax.ShapeDtypeStruct((2, L), jnp.float32),
      mesh=mesh,
      compiler_params=pltpu.CompilerParams(needs_layout_passes=False),
      scratch_types=[
          pltpu.VMEM((n_per_tile,), jnp.int32),
          pltpu.VMEM((n_per_tile,), jnp.float32),
          pltpu.VMEM((n_per_tile,), jnp.float32),
          pltpu.VMEM((n_per_tile,), jnp.float32),
          pltpu.VMEM((n_per_tile,), jnp.float32),
          pltpu.VMEM((L,), jnp.float32),
          pltpu.VMEM((num_subcores, L), jnp.float32),
          pltpu.VMEM_SHARED((n,), jnp.float32),
          pltpu.VMEM_SHARED((num_subcores, L), jnp.float32),
          pltpu.SemaphoreType.DMA,
      ],
  )
  def kernel(gt_idx, pr_idx, e_gt, e_pr, out,
             idx_v, ones_v, zro_v, cnt_v, e_v, res_v, acc_v,
             hist_sh, part_sh, sem):
    c = lax.axis_index("c")
    s = lax.axis_index("s")
    lanes = lax.iota(jnp.int32, L)

    def fill(ref, length, value):
      v = jnp.full((L,), value, jnp.float32)
      def body(g, carry):
        ref[pl.ds(g * L, L)] = v
        return carry
      lax.fori_loop(0, length // L, body, 0)

    def side(idx_hbm, e_hbm, frac):
      fill(zro_v, n_per_tile, 0.0)
      fill(ones_v, n_per_tile, 1.0)
      pltpu.sync_copy(idx_hbm.at[pl.ds(s * n_per_tile, n_per_tile)], idx_v)
      pltpu.sync_copy(zro_v, hist_sh.at[pl.ds(s * n_per_tile, n_per_tile)])
      plsc.subcore_barrier()
      pltpu.sync_copy(ones_v, hist_sh.at[idx_v], add=True)
      pltpu.sync_copy(e_hbm.at[pl.ds(s * n_per_tile, n_per_tile)], e_v)
      plsc.subcore_barrier()
      pltpu.async_copy(hist_sh.at[idx_v], cnt_v, sem).wait()

      def body(g, acc):
        cnt = cnt_v[pl.ds(g * L, L)]
        e = e_v[pl.ds(g * L, L)]
        cost = frac * e / (_sqrt16(cnt) + jnp.float32(N_EPS))
        return acc + jnp.abs(jnp.float32(1.0) - cost)

      acc = lax.fori_loop(0, groups, body, jnp.zeros((L,), jnp.float32))

      res_v[...] = acc
      pltpu.sync_copy(res_v, part_sh.at[s])
      plsc.subcore_barrier()

      @pl.when(s == 0)
      def _():
        pltpu.sync_copy(part_sh, acc_v)
        tot = acc_v[0, :]
        for i in range(1, num_subcores):
          tot = tot + acc_v[i, :]
        total = jnp.sum(tot) * inv_scale
        res_v[...] = jnp.where(lanes == 0, total, jnp.float32(0.0))
        pltpu.sync_copy(res_v, out.at[c])

    @pl.when(c == 0)
    def _():
      side(gt_idx, e_gt, jnp.float32(1.0))

    @pl.when(c == 1)
    def _():
      side(pr_idx, e_pr, jnp.float32(1.0))

  return kernel


@jax.jit
def kernel(gt_pts, gt_paired_pts, pred_pts, pred_paired_pts,
           gt_paired_idx, pred_paired_idx):
  n = gt_pts.shape[0]
  assert pred_pts.shape[0] == n
  e_gt, e_pr = _tc_exp_kernel(n, 2048)(
      gt_pts, gt_paired_pts, pred_pts, pred_paired_pts)
  out = _sc_loss_kernel(n, 16)(
      gt_paired_idx.astype(jnp.int32),
      pred_paired_idx.astype(jnp.int32),
      e_gt, e_pr,
  )
  return out[0, 0] + out[1, 0]


# speedup vs baseline: 2.1127x; 2.1127x over previous
import functools

import jax
import jax.numpy as jnp
from jax import lax
from jax.experimental import pallas as pl
from jax.experimental.pallas import tpu as pltpu
from jax.experimental.pallas import tpu_sc as plsc

ALPHA = 40.0
N_EPS = 1e-6
L = 16


def _sqrt16(x):
  xs = jnp.maximum(x, jnp.float32(1e-30))
  i = plsc.bitcast(xs, jnp.int32)
  i = jnp.int32(0x5F3759DF) - (i >> 1)
  y = plsc.bitcast(i, jnp.float32)
  for _ in range(3):
    y = y * (jnp.float32(1.5) - jnp.float32(0.5) * xs * y * y)
  return xs * y


def _sc_kernel(n, num_subcores):
  n_per_tile = n // num_subcores
  groups = n_per_tile // L
  mesh = plsc.VectorSubcoreMesh(
      core_axis_name="c", subcore_axis_name="s",
      num_cores=2, num_subcores=num_subcores)
  inv_scale = jnp.float32(0.5 / n)

  @functools.partial(
      pl.kernel,
      out_type=jax.ShapeDtypeStruct((2, L), jnp.float32),
      mesh=mesh,
      compiler_params=pltpu.CompilerParams(needs_layout_passes=False),
      scratch_types=[
          pltpu.VMEM((n_per_tile,), jnp.int32),
          pltpu.VMEM((n_per_tile,), jnp.float32),
          pltpu.VMEM((n_per_tile,), jnp.float32),
          pltpu.VMEM((n_per_tile,), jnp.float32),
          pltpu.VMEM((n_per_tile,), jnp.float32),
          pltpu.VMEM((n_per_tile,), jnp.float32),
          pltpu.VMEM((n_per_tile,), jnp.float32),
          pltpu.VMEM((n_per_tile,), jnp.float32),
          pltpu.VMEM((n_per_tile,), jnp.float32),
          pltpu.VMEM((n_per_tile,), jnp.float32),
          pltpu.VMEM((L,), jnp.float32),
          pltpu.VMEM((num_subcores, L), jnp.float32),
          pltpu.VMEM_SHARED((n,), jnp.float32),
          pltpu.VMEM_SHARED((num_subcores, L), jnp.float32),
          pltpu.SemaphoreType.DMA,
      ],
  )
  def kernel(gt_xyz, gt_p_xyz, pr_xyz, pr_p_xyz, gt_idx, pr_idx, out,
             idx_v, ones_v, zro_v, cnt_v, ax_v, ay_v, az_v, bx_v, by_v, bz_v,
             res_v, acc_v, hist_sh, part_sh, sem):
    c = lax.axis_index("c")
    s = lax.axis_index("s")
    lanes = lax.iota(jnp.int32, L)

    def fill(ref, length, value):
      v = jnp.full((L,), value, jnp.float32)
      def body(g, carry):
        ref[pl.ds(g * L, L)] = v
        return carry
      lax.fori_loop(0, length // L, body, 0)

    def side(idx_hbm, a_hbm, b_hbm, frac):
      fill(zro_v, n_per_tile, 0.0)
      fill(ones_v, n_per_tile, 1.0)
      pltpu.sync_copy(idx_hbm.at[pl.ds(s * n_per_tile, n_per_tile)], idx_v)
      pltpu.sync_copy(zro_v, hist_sh.at[pl.ds(s * n_per_tile, n_per_tile)])
      plsc.subcore_barrier()
      pltpu.sync_copy(ones_v, hist_sh.at[idx_v], add=True)
      base = s * n_per_tile
      pltpu.sync_copy(a_hbm.at[pl.ds(base, n_per_tile)], ax_v)
      pltpu.sync_copy(a_hbm.at[pl.ds(n + base, n_per_tile)], ay_v)
      pltpu.sync_copy(a_hbm.at[pl.ds(2 * n + base, n_per_tile)], az_v)
      pltpu.sync_copy(b_hbm.at[pl.ds(base, n_per_tile)], bx_v)
      pltpu.sync_copy(b_hbm.at[pl.ds(n + base, n_per_tile)], by_v)
      pltpu.sync_copy(b_hbm.at[pl.ds(2 * n + base, n_per_tile)], bz_v)
      plsc.subcore_barrier()
      pltpu.async_copy(hist_sh.at[idx_v], cnt_v, sem).wait()

      def body(g, acc):
        sl = pl.ds(g * L, L)
        dx = ax_v[sl] - bx_v[sl]
        dy = ay_v[sl] - by_v[sl]
        dz = az_v[sl] - bz_v[sl]
        dist = _sqrt16(dx * dx + dy * dy + dz * dz)
        e = jnp.exp(-jnp.float32(ALPHA) * dist)
        cost = frac * e / (_sqrt16(cnt_v[sl]) + jnp.float32(N_EPS))
        return acc + jnp.abs(jnp.float32(1.0) - cost)

      acc = lax.fori_loop(0, groups, body, jnp.zeros((L,), jnp.float32))

      res_v[...] = acc
      pltpu.sync_copy(res_v, part_sh.at[s])
      plsc.subcore_barrier()

      @pl.when(s == 0)
      def _():
        pltpu.sync_copy(part_sh, acc_v)
        tot = acc_v[0, :]
        for i in range(1, num_subcores):
          tot = tot + acc_v[i, :]
        total = jnp.sum(tot) * inv_scale
        res_v[...] = jnp.where(lanes == 0, total, jnp.float32(0.0))
        pltpu.sync_copy(res_v, out.at[c])

    @pl.when(c == 0)
    def _():
      side(gt_idx, gt_xyz, gt_p_xyz, jnp.float32(1.0))

    @pl.when(c == 1)
    def _():
      side(pr_idx, pr_xyz, pr_p_xyz, jnp.float32(1.0))

  return kernel


@jax.jit
def kernel(gt_pts, gt_paired_pts, pred_pts, pred_paired_pts,
           gt_paired_idx, pred_paired_idx):
  n = gt_pts.shape[0]
  assert pred_pts.shape[0] == n
  out = _sc_kernel(n, 16)(
      gt_pts.T.reshape(-1),
      gt_paired_pts.T.reshape(-1),
      pred_pts.T.reshape(-1),
      pred_paired_pts.T.reshape(-1),
      gt_paired_idx.astype(jnp.int32),
      pred_paired_idx.astype(jnp.int32),
  )
  return out[0, 0] + out[1, 0]


# speedup vs baseline: 2.5189x; 1.1923x over previous
import functools

import jax
import jax.numpy as jnp
from jax import lax
from jax.experimental import pallas as pl
from jax.experimental.pallas import tpu as pltpu
from jax.experimental.pallas import tpu_sc as plsc

ALPHA = 40.0
N_EPS = 1e-6
L = 16


def _sqrt16(x):
  xs = jnp.maximum(x, jnp.float32(1e-30))
  i = plsc.bitcast(xs, jnp.int32)
  i = jnp.int32(0x5F3759DF) - (i >> 1)
  y = plsc.bitcast(i, jnp.float32)
  for _ in range(3):
    y = y * (jnp.float32(1.5) - jnp.float32(0.5) * xs * y * y)
  return xs * y


def _sc_kernel(n, num_subcores):
  n_per_tile = n // num_subcores
  groups = n_per_tile // L
  mesh = plsc.VectorSubcoreMesh(
      core_axis_name="c", subcore_axis_name="s",
      num_cores=2, num_subcores=num_subcores)
  inv_scale = jnp.float32(0.5 / n)

  @functools.partial(
      pl.kernel,
      out_type=jax.ShapeDtypeStruct((2, L), jnp.float32),
      mesh=mesh,
      compiler_params=pltpu.CompilerParams(needs_layout_passes=False),
      scratch_types=[
          pltpu.VMEM((n_per_tile,), jnp.int32),
          pltpu.VMEM((n_per_tile,), jnp.float32),
          pltpu.VMEM((n_per_tile,), jnp.float32),
          pltpu.VMEM((n_per_tile,), jnp.float32),
          pltpu.VMEM((6, n_per_tile), jnp.float32),
          pltpu.VMEM((L,), jnp.float32),
          pltpu.VMEM((num_subcores, L), jnp.float32),
          pltpu.VMEM_SHARED((n,), jnp.float32),
          pltpu.VMEM_SHARED((num_subcores, L), jnp.float32),
          pltpu.SemaphoreType.DMA,
          pltpu.SemaphoreType.DMA,
      ],
  )
  def kernel(pts_all, gt_idx, pr_idx, out,
             idx_v, ones_v, zro_v, cnt_v, xyz_v,
             res_v, acc_v, hist_sh, part_sh, sem, sem2):
    c = lax.axis_index("c")
    s = lax.axis_index("s")
    lanes = lax.iota(jnp.int32, L)

    def fill(ref, length, value):
      v = jnp.full((L,), value, jnp.float32)
      def body(g, carry):
        ref[pl.ds(g * L, L)] = v
        return carry
      lax.fori_loop(0, length // L, body, 0)

    def side(idx_hbm, a_off, frac):
      base = s * n_per_tile
      idx_dma = pltpu.async_copy(
          idx_hbm.at[pl.ds(base, n_per_tile)], idx_v, sem)
      coord_dmas = [
          pltpu.async_copy(
              pts_all.at[pl.ds(a_off + p * n + base, n_per_tile)],
              xyz_v.at[p], sem2)
          for p in range(6)
      ]
      fill(zro_v, n_per_tile, 0.0)
      fill(ones_v, n_per_tile, 1.0)
      pltpu.sync_copy(zro_v, hist_sh.at[pl.ds(base, n_per_tile)])
      idx_dma.wait()
      plsc.subcore_barrier()
      pltpu.sync_copy(ones_v, hist_sh.at[idx_v], add=True)
      plsc.subcore_barrier()
      gat = pltpu.async_copy(hist_sh.at[idx_v], cnt_v, sem)
      for d in coord_dmas:
        d.wait()
      gat.wait()

      def body(g, acc):
        sl = pl.ds(g * L, L)
        dx = xyz_v[0, sl] - xyz_v[3, sl]
        dy = xyz_v[1, sl] - xyz_v[4, sl]
        dz = xyz_v[2, sl] - xyz_v[5, sl]
        dist = _sqrt16(dx * dx + dy * dy + dz * dz)
        e = jnp.exp(-jnp.float32(ALPHA) * dist)
        cost = frac * e / (_sqrt16(cnt_v[sl]) + jnp.float32(N_EPS))
        return acc + jnp.abs(jnp.float32(1.0) - cost)

      acc = lax.fori_loop(0, groups, body, jnp.zeros((L,), jnp.float32))

      res_v[...] = acc
      pltpu.sync_copy(res_v, part_sh.at[s])
      plsc.subcore_barrier()

      @pl.when(s == 0)
      def _():
        pltpu.sync_copy(part_sh, acc_v)
        tot = acc_v[0, :]
        for i in range(1, num_subcores):
          tot = tot + acc_v[i, :]
        total = jnp.sum(tot) * inv_scale
        res_v[...] = jnp.where(lanes == 0, total, jnp.float32(0.0))
        pltpu.sync_copy(res_v, out.at[c])

    @pl.when(c == 0)
    def _():
      side(gt_idx, 0, jnp.float32(1.0))

    @pl.when(c == 1)
    def _():
      side(pr_idx, 6 * n, jnp.float32(1.0))

  return kernel


@jax.jit
def kernel(gt_pts, gt_paired_pts, pred_pts, pred_paired_pts,
           gt_paired_idx, pred_paired_idx):
  n = gt_pts.shape[0]
  assert pred_pts.shape[0] == n
  pts_all = jnp.concatenate([
      gt_pts.T.reshape(-1),
      gt_paired_pts.T.reshape(-1),
      pred_pts.T.reshape(-1),
      pred_paired_pts.T.reshape(-1),
  ])
  out = _sc_kernel(n, 16)(
      pts_all,
      gt_paired_idx.astype(jnp.int32),
      pred_paired_idx.astype(jnp.int32),
  )
  return out[0, 0] + out[1, 0]
